# baseline (speedup 1.0000x reference)
import functools

import jax
import jax.numpy as jnp
from jax import lax
from jax.experimental import pallas as pl
from jax.experimental.pallas import tpu as pltpu
from jax.experimental.pallas import tpu_sc as plsc

N = 10000
NP = 10112
E = 320000
NW = 32
CHUNK = 128
EPT = 10240
NCHUNK = EPT // CHUNK
EP = NW * EPT
RPT = NP // 16
F = 16

_mesh = plsc.VectorSubcoreMesh(core_axis_name="c", subcore_axis_name="s")


NBUF = 8
DLT = 4

CONST_ROWS = RPT + CHUNK


@functools.partial(
    pl.kernel,
    mesh=_mesh,
    out_type=jax.ShapeDtypeStruct((2 * NP, F), jnp.float32),
    compiler_params=pltpu.CompilerParams(use_tc_tiling_on_sc=False),
    scratch_types=[
        pltpu.VMEM((NCHUNK, CHUNK), jnp.int32),
        pltpu.VMEM((NCHUNK, CHUNK), jnp.int32),
        pltpu.VMEM((NBUF, CHUNK, F), jnp.float32),
        pltpu.VMEM_SHARED((NP, F), jnp.float32),
    ] + [pltpu.SemaphoreType.DMA] * (2 * NBUF),
)
def _sc_agg(u_hbm, src_hbm, dst_hbm, cz_hbm, out_hbm, src_v, dst_v, gb, acc,
            *sems):
    cid = lax.axis_index("c")
    sid = lax.axis_index("s")
    w = cid * 16 + sid

    @pl.when(cid == 0)
    def _():
        pltpu.sync_copy(u_hbm.at[pl.ds(sid * RPT, RPT)],
                        acc.at[pl.ds(sid * RPT, RPT)])

    @pl.when(cid == 1)
    def _():
        pltpu.sync_copy(cz_hbm.at[pl.ds(0, RPT)],
                        acc.at[pl.ds(sid * RPT, RPT)])

    pltpu.sync_copy(src_hbm.at[pl.ds(w * NCHUNK, NCHUNK)], src_v)
    pltpu.sync_copy(dst_hbm.at[pl.ds(w * NCHUNK, NCHUNK)], dst_v)
    plsc.subcore_barrier()

    gsems, ssems = sems[:NBUF], sems[NBUF:]

    def fire(c, b):
        pltpu.async_copy(u_hbm.at[src_v.at[c]], gb.at[b], gsems[b])

    for b in range(DLT):
        fire(b, b)

    def outer(t, carry):
        for b in range(NBUF):
            c = t * NBUF + b
            b2 = (b + DLT) % NBUF
            pltpu.make_async_copy(u_hbm.at[src_v.at[c]], gb.at[b],
                                  gsems[b]).wait()
            pltpu.async_copy(gb.at[b], acc.at[dst_v.at[c]], ssems[b],
                             add=True)

            @pl.when(c + DLT - NBUF >= 0)
            def _():
                pltpu.make_async_copy(
                    gb.at[b2], acc.at[dst_v.at[c + DLT - NBUF]],
                    ssems[b2]).wait()

            @pl.when(c + DLT < NCHUNK)
            def _():
                fire(c + DLT, b2)
        return carry
    lax.fori_loop(0, NCHUNK // NBUF, outer, 0)

    for b in range(NBUF - DLT):
        bb = DLT + b
        pltpu.make_async_copy(gb.at[bb],
                              acc.at[dst_v.at[NCHUNK - NBUF + bb]],
                              ssems[bb]).wait()

    plsc.subcore_barrier()
    pltpu.sync_copy(acc.at[pl.ds(sid * RPT, RPT)],
                    out_hbm.at[pl.ds(cid * NP + sid * RPT, RPT)])


@functools.partial(
    pl.kernel,
    mesh=_mesh,
    out_type=jax.ShapeDtypeStruct((2 * NP, F), jnp.float32),
    compiler_params=pltpu.CompilerParams(use_tc_tiling_on_sc=False),
    scratch_types=[
        pltpu.VMEM((NCHUNK, CHUNK), jnp.int32),
        pltpu.VMEM((CHUNK, F), jnp.float32),
        pltpu.VMEM_SHARED((NP, F), jnp.float32),
    ] + [pltpu.SemaphoreType.DMA] * NBUF,
)
def _sc_deg(dst_hbm, cz_hbm, out_hbm, dst_v, ones_v, acc, *ssems):
    cid = lax.axis_index("c")
    sid = lax.axis_index("s")
    w = cid * 16 + sid

    pltpu.sync_copy(cz_hbm.at[pl.ds(0, RPT)], acc.at[pl.ds(sid * RPT, RPT)])
    pltpu.sync_copy(cz_hbm.at[pl.ds(RPT, CHUNK)], ones_v)
    pltpu.sync_copy(dst_hbm.at[pl.ds(w * NCHUNK, NCHUNK)], dst_v)
    plsc.subcore_barrier()

    def body(t, carry):
        for b in range(NBUF):
            c = t * NBUF + b

            @pl.when(c - NBUF >= 0)
            def _():
                pltpu.make_async_copy(ones_v, acc.at[dst_v.at[c - NBUF]],
                                      ssems[b]).wait()
            pltpu.async_copy(ones_v, acc.at[dst_v.at[c]], ssems[b], add=True)
        return carry
    lax.fori_loop(0, NCHUNK // NBUF, body, 0)

    for b in range(NBUF):
        pltpu.make_async_copy(ones_v, acc.at[dst_v.at[NCHUNK - NBUF + b]],
                              ssems[b]).wait()

    plsc.subcore_barrier()
    pltpu.sync_copy(acc.at[pl.ds(sid * RPT, RPT)],
                    out_hbm.at[pl.ds(cid * NP + sid * RPT, RPT)])


NP8 = NP // 8
NV8 = N // 8


def _tc_mm0_body(x3_ref, w_ref, o_ref):
    cols = [jnp.dot(x3_ref[:, g, :], w_ref[...],
                    preferred_element_type=jnp.float32) for g in range(8)]
    o_ref[...] = jnp.concatenate(cols, axis=1)


def _tc_head_body(dp_ref, m_ref, u_ref, s_ref):
    s = lax.rsqrt(dp_ref[0] + dp_ref[1] + 1.0)
    s_ref[...] = s
    u_ref[...] = s * m_ref[...]


def _tc_mid_body(p_ref, s_ref, b_ref, bd_ref, o_ref):
    s = s_ref[...]
    y = jnp.maximum(s * (p_ref[0] + p_ref[1]) + b_ref[...], 0.0)
    un = s * jnp.dot(y, bd_ref[...], preferred_element_type=jnp.float32)
    row = lax.broadcasted_iota(jnp.int32, (NP8, 128), 0)
    o_ref[...] = jnp.where(row < NV8, un, 0.0)


def _tc_mid3_body(p_ref, s_ref, b_ref, o_ref):
    s = s_ref[...]
    y = jnp.maximum(s * (p_ref[0] + p_ref[1]) + b_ref[...], 0.0)
    row = lax.broadcasted_iota(jnp.int32, (NP8, 128), 0)
    o_ref[...] = jnp.where(row < NV8, s * y, 0.0)


def _tc_tail_body(p_ref, s_ref, w_ref, b_ref, o_ref):
    g = s_ref[...] * (p_ref[0] + p_ref[1])
    for gg in range(8):
        o_ref[:, gg, :] = jnp.dot(
            g[:, 16 * gg:16 * (gg + 1)], w_ref[...],
            preferred_element_type=jnp.float32) + b_ref[...]


_f32 = jnp.float32
_tc_mm0 = pl.pallas_call(
    _tc_mm0_body, out_shape=jax.ShapeDtypeStruct((NP8, 128), _f32))
_tc_head = pl.pallas_call(
    _tc_head_body,
    out_shape=(jax.ShapeDtypeStruct((NP8, 128), _f32),
               jax.ShapeDtypeStruct((NP8, 128), _f32)))
_tc_mid = pl.pallas_call(
    _tc_mid_body, out_shape=jax.ShapeDtypeStruct((NP8, 128), _f32))
_tc_mid3 = pl.pallas_call(
    _tc_mid3_body, out_shape=jax.ShapeDtypeStruct((NP8, 128), _f32))
_tc_tail = pl.pallas_call(
    _tc_tail_body, out_shape=jax.ShapeDtypeStruct((NP8, 8, 128), _f32))


def kernel(x, edge_index, W0, b0, W1, b1, W2, b2, W3, b3):
    src = edge_index[0]
    dst = edge_index[1]
    npad = EP - E
    pad_idx = N + (jnp.arange(npad, dtype=jnp.int32) % (NP - N))
    src_p = jnp.concatenate([src, pad_idx]).reshape(NW * NCHUNK, CHUNK)
    dst_p = jnp.concatenate([dst, pad_idx]).reshape(NW * NCHUNK, CHUNK)
    x3 = jnp.pad(x, ((0, NP - N), (0, 0))).reshape(NP8, 8, 128)
    eye8 = jnp.eye(8, dtype=_f32)
    bd1 = jnp.kron(eye8, W1)
    bd2 = jnp.kron(eye8, W2)
    b0r = jnp.tile(b0, 8).reshape(1, 128)
    b1r = jnp.tile(b1, 8).reshape(1, 128)
    b2r = jnp.tile(b2, 8).reshape(1, 128)
    b3r = b3.reshape(1, 128)

    cz = jnp.concatenate([jnp.zeros((RPT, F), _f32),
                          jnp.ones((CHUNK, F), _f32)])

    def pk(parts):
        return parts.reshape(2, NP8, 128)

    m0 = _tc_mm0(x3, W0)
    dparts = _sc_deg(dst_p, cz)
    u0, s16 = _tc_head(pk(dparts), m0)
    p0 = _sc_agg(u0.reshape(NP, F), src_p, dst_p, cz)
    u1 = _tc_mid(pk(p0), s16, b0r, bd1)
    p1 = _sc_agg(u1.reshape(NP, F), src_p, dst_p, cz)
    u2 = _tc_mid(pk(p1), s16, b1r, bd2)
    p2 = _sc_agg(u2.reshape(NP, F), src_p, dst_p, cz)
    u3 = _tc_mid3(pk(p2), s16, b2r)
    p3 = _sc_agg(u3.reshape(NP, F), src_p, dst_p, cz)
    out = _tc_tail(pk(p3), s16, W3, b3r)
    return out.reshape(NP, 128)[:N]


# speedup vs baseline: 1.0979x; 1.0979x over previous
import functools

import jax
import jax.numpy as jnp
from jax import lax
from jax.experimental import pallas as pl
from jax.experimental.pallas import tpu as pltpu
from jax.experimental.pallas import tpu_sc as plsc

N = 10000
NP = 10112
E = 320000
NW = 32
CHUNK = 128
EPT = 10240
NCHUNK = EPT // CHUNK
EP = NW * EPT
RPT = NP // 16
F = 16

_mesh = plsc.VectorSubcoreMesh(core_axis_name="c", subcore_axis_name="s")


NBUF = 8
DLT = 4

CONST_ROWS = RPT + CHUNK


@functools.partial(
    pl.kernel,
    mesh=_mesh,
    out_type=jax.ShapeDtypeStruct((2 * NP, F), jnp.float32),
    compiler_params=pltpu.CompilerParams(use_tc_tiling_on_sc=False),
    scratch_types=[
        pltpu.VMEM((NCHUNK, CHUNK), jnp.int32),
        pltpu.VMEM((NCHUNK, CHUNK), jnp.int32),
        pltpu.VMEM((NBUF, CHUNK, F), jnp.float32),
        pltpu.VMEM_SHARED((NP, F), jnp.float32),
    ] + [pltpu.SemaphoreType.DMA] * NBUF,
)
def _sc_agg(u_hbm, src_hbm, dst_hbm, cz_hbm, out_hbm, src_v, dst_v, gb, acc,
            *sems):
    cid = lax.axis_index("c")
    sid = lax.axis_index("s")
    w = cid * 16 + sid

    @pl.when(cid == 0)
    def _():
        pltpu.sync_copy(u_hbm.at[pl.ds(sid * RPT, RPT)],
                        acc.at[pl.ds(sid * RPT, RPT)])

    @pl.when(cid == 1)
    def _():
        pltpu.sync_copy(cz_hbm.at[pl.ds(0, RPT)],
                        acc.at[pl.ds(sid * RPT, RPT)])

    pltpu.sync_copy(src_hbm.at[pl.ds(w * NCHUNK, NCHUNK)], src_v)
    pltpu.sync_copy(dst_hbm.at[pl.ds(w * NCHUNK, NCHUNK)], dst_v)
    plsc.subcore_barrier()

    def fire(c, b):
        pltpu.async_copy(u_hbm.at[src_v.at[c]], gb.at[b], sems[b])

    for b in range(NBUF):
        fire(b, b)

    def outer(t, carry):
        for b in range(NBUF):
            c = t * NBUF + b
            pltpu.make_async_copy(u_hbm.at[src_v.at[c]], gb.at[b],
                                  sems[b]).wait()
            pltpu.sync_copy(gb.at[b], acc.at[dst_v.at[c]], add=True)

            @pl.when(t < NCHUNK // NBUF - 1)
            def _():
                fire(c + NBUF, b)
        return carry
    lax.fori_loop(0, NCHUNK // NBUF, outer, 0)

    plsc.subcore_barrier()
    pltpu.sync_copy(acc.at[pl.ds(sid * RPT, RPT)],
                    out_hbm.at[pl.ds(cid * NP + sid * RPT, RPT)])


@functools.partial(
    pl.kernel,
    mesh=_mesh,
    out_type=jax.ShapeDtypeStruct((2 * NP, F), jnp.float32),
    compiler_params=pltpu.CompilerParams(use_tc_tiling_on_sc=False),
    scratch_types=[
        pltpu.VMEM((NCHUNK, CHUNK), jnp.int32),
        pltpu.VMEM((CHUNK, F), jnp.float32),
        pltpu.VMEM_SHARED((NP, F), jnp.float32),
    ],
)
def _sc_deg(dst_hbm, cz_hbm, out_hbm, dst_v, ones_v, acc):
    cid = lax.axis_index("c")
    sid = lax.axis_index("s")
    w = cid * 16 + sid

    pltpu.sync_copy(cz_hbm.at[pl.ds(0, RPT)], acc.at[pl.ds(sid * RPT, RPT)])
    pltpu.sync_copy(cz_hbm.at[pl.ds(RPT, CHUNK)], ones_v)
    pltpu.sync_copy(dst_hbm.at[pl.ds(w * NCHUNK, NCHUNK)], dst_v)
    plsc.subcore_barrier()

    def body(c, carry):
        pltpu.sync_copy(ones_v, acc.at[dst_v.at[c]], add=True)
        return carry
    lax.fori_loop(0, NCHUNK, body, 0)

    plsc.subcore_barrier()
    pltpu.sync_copy(acc.at[pl.ds(sid * RPT, RPT)],
                    out_hbm.at[pl.ds(cid * NP + sid * RPT, RPT)])


NP8 = NP // 8
NV8 = N // 8


def _tc_mm0_body(x3_ref, w_ref, o_ref):
    cols = [jnp.dot(x3_ref[:, g, :], w_ref[...],
                    preferred_element_type=jnp.float32) for g in range(8)]
    o_ref[...] = jnp.concatenate(cols, axis=1)


def _tc_head_body(dp_ref, m_ref, u_ref, s_ref):
    s = lax.rsqrt(dp_ref[0] + dp_ref[1] + 1.0)
    s_ref[...] = s
    u_ref[...] = s * m_ref[...]


def _tc_mid_body(p_ref, s_ref, b_ref, bd_ref, o_ref):
    s = s_ref[...]
    y = jnp.maximum(s * (p_ref[0] + p_ref[1]) + b_ref[...], 0.0)
    un = s * jnp.dot(y, bd_ref[...], preferred_element_type=jnp.float32)
    row = lax.broadcasted_iota(jnp.int32, (NP8, 128), 0)
    o_ref[...] = jnp.where(row < NV8, un, 0.0)


def _tc_mid3_body(p_ref, s_ref, b_ref, o_ref):
    s = s_ref[...]
    y = jnp.maximum(s * (p_ref[0] + p_ref[1]) + b_ref[...], 0.0)
    row = lax.broadcasted_iota(jnp.int32, (NP8, 128), 0)
    o_ref[...] = jnp.where(row < NV8, s * y, 0.0)


def _tc_tail_body(p_ref, s_ref, w_ref, b_ref, o_ref):
    g = s_ref[...] * (p_ref[0] + p_ref[1])
    for gg in range(8):
        o_ref[:, gg, :] = jnp.dot(
            g[:, 16 * gg:16 * (gg + 1)], w_ref[...],
            preferred_element_type=jnp.float32) + b_ref[...]


_f32 = jnp.float32
_tc_mm0 = pl.pallas_call(
    _tc_mm0_body, out_shape=jax.ShapeDtypeStruct((NP8, 128), _f32))
_tc_head = pl.pallas_call(
    _tc_head_body,
    out_shape=(jax.ShapeDtypeStruct((NP8, 128), _f32),
               jax.ShapeDtypeStruct((NP8, 128), _f32)))
_tc_mid = pl.pallas_call(
    _tc_mid_body, out_shape=jax.ShapeDtypeStruct((NP8, 128), _f32))
_tc_mid3 = pl.pallas_call(
    _tc_mid3_body, out_shape=jax.ShapeDtypeStruct((NP8, 128), _f32))
_tc_tail = pl.pallas_call(
    _tc_tail_body, out_shape=jax.ShapeDtypeStruct((NP8, 8, 128), _f32))


def kernel(x, edge_index, W0, b0, W1, b1, W2, b2, W3, b3):
    src = edge_index[0]
    dst = edge_index[1]
    npad = EP - E
    pad_idx = N + (jnp.arange(npad, dtype=jnp.int32) % (NP - N))
    src_p = jnp.concatenate([src, pad_idx]).reshape(NW * NCHUNK, CHUNK)
    dst_p = jnp.concatenate([dst, pad_idx]).reshape(NW * NCHUNK, CHUNK)
    x3 = jnp.pad(x, ((0, NP - N), (0, 0))).reshape(NP8, 8, 128)
    eye8 = jnp.eye(8, dtype=_f32)
    bd1 = jnp.kron(eye8, W1)
    bd2 = jnp.kron(eye8, W2)
    b0r = jnp.tile(b0, 8).reshape(1, 128)
    b1r = jnp.tile(b1, 8).reshape(1, 128)
    b2r = jnp.tile(b2, 8).reshape(1, 128)
    b3r = b3.reshape(1, 128)

    cz = jnp.concatenate([jnp.zeros((RPT, F), _f32),
                          jnp.ones((CHUNK, F), _f32)])

    def pk(parts):
        return parts.reshape(2, NP8, 128)

    m0 = _tc_mm0(x3, W0)
    dparts = _sc_deg(dst_p, cz)
    u0, s16 = _tc_head(pk(dparts), m0)
    p0 = _sc_agg(u0.reshape(NP, F), src_p, dst_p, cz)
    u1 = _tc_mid(pk(p0), s16, b0r, bd1)
    p1 = _sc_agg(u1.reshape(NP, F), src_p, dst_p, cz)
    u2 = _tc_mid(pk(p1), s16, b1r, bd2)
    p2 = _sc_agg(u2.reshape(NP, F), src_p, dst_p, cz)
    u3 = _tc_mid3(pk(p2), s16, b2r)
    p3 = _sc_agg(u3.reshape(NP, F), src_p, dst_p, cz)
    out = _tc_tail(pk(p3), s16, W3, b3r)
    return out.reshape(NP, 128)[:N]
